# baseline (speedup 1.0000x reference)
import functools

import jax
import jax.numpy as jnp
from jax import lax
from jax.experimental import pallas as pl
from jax.experimental.pallas import tpu as pltpu
from jax.experimental.pallas import tpu_sc as plsc

_NU, _NB, _NI, _D = 50000, 20000, 100000, 64
_L = 2

_NC, _NS, _VL = 2, 16, 16
_NW = _NC * _NS
_GB = 128
_BB = 512
_GPB = _BB // _GB
_HC = 8
_R = 400


def _div_le(n, cap):
    for k in range(1, n + 1):
        if n % k == 0 and n // k <= cap:
            return n // k
    return 1


def _mesh():
    return plsc.VectorSubcoreMesh(core_axis_name="c", subcore_axis_name="s")


@functools.lru_cache(maxsize=None)
def _hist_fn(epad, nreal, npad):
    nbatch = epad // (_NW * _BB)
    sub = npad // _NS
    zb = _div_le(sub, 512)

    @functools.partial(
        pl.kernel,
        mesh=_mesh(),
        out_type=jax.ShapeDtypeStruct((_NC, npad, _HC), jnp.float32),
        scratch_types=[
            pltpu.VMEM((_BB,), jnp.int32),
            pltpu.VMEM((_GPB, _GB), jnp.int32),
            pltpu.VMEM((_GB, _HC), jnp.float32),
            pltpu.VMEM((zb, _HC), jnp.float32),
            pltpu.VMEM_SHARED((npad, _HC), jnp.float32),
            pltpu.SemaphoreType.DMA,
        ],
        compiler_params=pltpu.CompilerParams(use_tc_tiling_on_sc=False),
    )
    def hist(idx_hbm, ones_hbm, zeros_hbm, out_hbm,
             idx_v, lidx_v, ones_v, wb_v, cnt_sh, sem):
        del sem
        cid = lax.axis_index("c")
        sid = lax.axis_index("s")
        wid = sid * _NC + cid
        pltpu.sync_copy(ones_hbm, ones_v)
        pltpu.sync_copy(zeros_hbm, wb_v)
        for z in range(sub // zb):
            pltpu.sync_copy(wb_v, cnt_sh.at[pl.ds(sid * sub + z * zb, zb)])
        plsc.subcore_barrier()

        def batch(bi, c):
            off = (wid * nbatch + bi) * _BB
            pltpu.sync_copy(idx_hbm.at[pl.ds(off, _BB)], idx_v)
            for j in range(_BB // _VL):
                v = idx_v[pl.ds(j * _VL, _VL)]
                lv = jnp.where(v < nreal, v, nreal + (v & (_GB - 1)))
                lidx_v[j // (_GB // _VL), pl.ds((j % (_GB // _VL)) * _VL, _VL)] = lv
            for g in range(_GPB):
                pltpu.sync_copy(ones_v, cnt_sh.at[lidx_v.at[g]], add=True)
            return c

        lax.fori_loop(0, nbatch, batch, 0)
        plsc.subcore_barrier()
        for z in range(sub // zb):
            r0 = sid * sub + z * zb
            pltpu.sync_copy(cnt_sh.at[pl.ds(r0, zb)], wb_v)
            pltpu.sync_copy(wb_v, out_hbm.at[cid, pl.ds(r0, zb)])

    return hist


def _hist(dst_flat, nreal, epad):
    npad = -(-(nreal + _GB) // (_NS * _GB)) * (_NS * _GB)
    sub = npad // _NS
    zb = _div_le(sub, 512)
    ones = jnp.zeros((_GB, _HC), jnp.float32).at[:, 0].set(1.0)
    zeros = jnp.zeros((zb, _HC), jnp.float32)
    out = _hist_fn(epad, nreal, npad)(dst_flat, ones, zeros)
    return out[:, :nreal, 0]


_IC = 1024
_ICG = _IC // _GB
_PB = 512
_STG = _IC + _VL
_DUMMY = 1 << 29


def _cap(epad):
    return -(-(epad // _NW) // _IC) * _IC


@functools.lru_cache(maxsize=None)
def _part_fn(epad, split):
    epw = epad // _NW
    cap = _cap(epad)
    nchk = epw // _PB
    ntot = _NC * _NW * cap

    @functools.partial(
        pl.kernel,
        mesh=_mesh(),
        out_type=[
            jax.ShapeDtypeStruct((ntot,), jnp.int32),
            jax.ShapeDtypeStruct((ntot,), jnp.int32),
            jax.ShapeDtypeStruct((_NW, _VL), jnp.int32),
        ],
        scratch_types=[
            pltpu.VMEM((_PB,), jnp.int32),
            pltpu.VMEM((_PB,), jnp.int32),
            pltpu.VMEM((_STG,), jnp.int32),
            pltpu.VMEM((_STG,), jnp.int32),
            pltpu.VMEM((_STG,), jnp.int32),
            pltpu.VMEM((_STG,), jnp.int32),
            pltpu.VMEM((_VL,), jnp.int32),
            pltpu.SemaphoreType.DMA,
        ],
        compiler_params=pltpu.CompilerParams(use_tc_tiling_on_sc=False,
                                             needs_layout_passes=False),
    )
    def part(src_hbm, dst_hbm, psrc_hbm, pdst_hbm, cnt_hbm,
             src_v, dst_v, ss0, sd0, ss1, sd1, cnt_v, sem):
        del sem
        cid = lax.axis_index("c")
        sid = lax.axis_index("s")
        wid = sid * _NC + cid
        e0 = wid * epw
        lanes = lax.broadcasted_iota(jnp.int32, (_VL,), 0)

        def flush(bkt, ss, sd, cur, nf):
            regbase = (bkt * _NW + wid) * cap
            pltpu.sync_copy(ss.at[pl.ds(0, _IC)],
                            psrc_hbm.at[pl.ds(regbase + nf * _IC, _IC)])
            pltpu.sync_copy(sd.at[pl.ds(0, _IC)],
                            pdst_hbm.at[pl.ds(regbase + nf * _IC, _IC)])
            r = cur - _IC
            vs = jnp.where(lanes < r, ss[pl.ds(_IC, _VL)], 0)
            vd = jnp.where(lanes < r, sd[pl.ds(_IC, _VL)], _DUMMY)
            ss[pl.ds(0, _VL)] = vs
            sd[pl.ds(0, _VL)] = vd
            return r, nf + 1

        def maybe_flush(bkt, ss, sd, cur, nf):
            return lax.cond(
                cur >= _IC,
                lambda c, n: flush(bkt, ss, sd, c, n),
                lambda c, n: (c, n),
                cur, nf)

        def chunkbody(ki, carry):
            cur0, cur1, nf0, nf1 = carry
            off = e0 + ki * _PB
            pltpu.sync_copy(src_hbm.at[pl.ds(off, _PB)], src_v)
            pltpu.sync_copy(dst_hbm.at[pl.ds(off, _PB)], dst_v)
            for j in range(_PB // _VL):
                vs = src_v[pl.ds(j * _VL, _VL)]
                vd = dst_v[pl.ds(j * _VL, _VL)]
                m0 = vd < split
                m1 = (vd >= split) & (vd < _DUMMY)
                key0 = jnp.where(m0, 0, _VL) + lanes
                key1 = jnp.where(m1, 0, _VL) + lanes
                _, s0v = plsc.sort_key_val(key0, vs)
                _, d0v = plsc.sort_key_val(key0, vd)
                ss0[pl.ds(cur0, _VL)] = s0v
                sd0[pl.ds(cur0, _VL)] = d0v
                cur0 = cur0 + jnp.sum(m0.astype(jnp.int32))
                cur0, nf0 = maybe_flush(0, ss0, sd0, cur0, nf0)
                _, s1v = plsc.sort_key_val(key1, vs)
                _, d1v = plsc.sort_key_val(key1, vd)
                ss1[pl.ds(cur1, _VL)] = s1v
                sd1[pl.ds(cur1, _VL)] = d1v
                cur1 = cur1 + jnp.sum(m1.astype(jnp.int32))
                cur1, nf1 = maybe_flush(1, ss1, sd1, cur1, nf1)
            return cur0, cur1, nf0, nf1

        cur0, cur1, nf0, nf1 = lax.fori_loop(
            0, nchk, chunkbody, (jnp.int32(0), jnp.int32(0),
                                 jnp.int32(0), jnp.int32(0)))

        for g in range(_STG // _VL):
            k0 = cur0 - g * _VL
            ss0[pl.ds(g * _VL, _VL)] = jnp.where(
                lanes < k0, ss0[pl.ds(g * _VL, _VL)], 0)
            sd0[pl.ds(g * _VL, _VL)] = jnp.where(
                lanes < k0, sd0[pl.ds(g * _VL, _VL)], _DUMMY)
            k1 = cur1 - g * _VL
            ss1[pl.ds(g * _VL, _VL)] = jnp.where(
                lanes < k1, ss1[pl.ds(g * _VL, _VL)], 0)
            sd1[pl.ds(g * _VL, _VL)] = jnp.where(
                lanes < k1, sd1[pl.ds(g * _VL, _VL)], _DUMMY)
        _, nf0 = lax.cond(
            cur0 > 0,
            lambda c, n: flush(0, ss0, sd0, c, n),
            lambda c, n: (c, n), cur0 + _IC, nf0)
        _, nf1 = lax.cond(
            cur1 > 0,
            lambda c, n: flush(1, ss1, sd1, c, n),
            lambda c, n: (c, n), cur1 + _IC, nf1)

        cnt_v[pl.ds(0, _VL)] = (jnp.where(lanes == 0, nf0, 0)
                                + jnp.where(lanes == 1, nf1, 0))
        pltpu.sync_copy(cnt_v, cnt_hbm.at[wid])

    return part


def _part(src_flat, dst_flat, nd, epad):
    chunk, nch = _chunk_cfg(nd)
    split = (nch // _NC) * chunk
    psrc, pdst, cnt = _part_fn(epad, split)(src_flat, dst_flat)
    return psrc, pdst, cnt.reshape(-1)


@functools.lru_cache(maxsize=None)
def _spmm_fn(epad, chunk, nch):
    cap = _cap(epad)
    ch_per_core = nch // _NC
    sub_rows = chunk // _NS

    @functools.partial(
        pl.kernel,
        mesh=_mesh(),
        out_type=jax.ShapeDtypeStruct((nch * chunk, _D), jnp.float32),
        scratch_types=[
            pltpu.VMEM((_IC,), jnp.int32),
            pltpu.VMEM((_IC,), jnp.int32),
            pltpu.VMEM((_ICG, _GB), jnp.int32),
            pltpu.VMEM((2, _GB, _D), jnp.float32),
            pltpu.VMEM((_NW * _VL + _VL,), jnp.int32),
            pltpu.VMEM_SHARED((chunk + _GB, _D), jnp.float32),
            pltpu.SemaphoreType.DMA,
        ],
        compiler_params=pltpu.CompilerParams(use_tc_tiling_on_sc=False),
    )
    def spmm(psrc_hbm, pdst_hbm, cnt_hbm, feat_hbm, zeros_hbm, out_hbm,
             sidx_v, didx_v, lidx_v, ring_v, cnt_v, acc_sh, sem):
        cid = lax.axis_index("c")
        sid = lax.axis_index("s")
        pltpu.sync_copy(cnt_hbm, cnt_v.at[pl.ds(0, _NW * _VL)])
        for ci in range(ch_per_core):
            chunk_id = cid * ch_per_core + ci
            base = chunk_id * chunk
            pltpu.sync_copy(zeros_hbm, ring_v.at[0])
            for z in range(sub_rows // _GB):
                pltpu.sync_copy(
                    ring_v.at[0],
                    acc_sh.at[pl.ds(sid * sub_rows + z * _GB, _GB)])
            plsc.subcore_barrier()

            for reg in range(_NW // _NS):
                w = sid * (_NW // _NS) + reg
                nb = cnt_v[pl.ds(w * _VL + cid, _VL)][0]
                regbase = (cid * _NW + w) * cap

                def batch(bi, c):
                    off = regbase + bi * _IC
                    pltpu.sync_copy(psrc_hbm.at[pl.ds(off, _IC)], sidx_v)
                    pltpu.sync_copy(pdst_hbm.at[pl.ds(off, _IC)], didx_v)
                    cps = [pltpu.async_copy(
                        feat_hbm.at[sidx_v.at[pl.ds(0, _GB)]],
                        ring_v.at[0], sem)]
                    for j in range(_IC // _VL):
                        v = didx_v[pl.ds(j * _VL, _VL)]
                        loc = v - base
                        ok = (loc >= 0) & (loc < chunk)
                        lv = jnp.where(ok, loc, chunk + (v & (_GB - 1)))
                        lidx_v[j // (_GB // _VL),
                               pl.ds((j % (_GB // _VL)) * _VL, _VL)] = lv
                    for g in range(_ICG):
                        if g + 1 < _ICG:
                            cps.append(pltpu.async_copy(
                                feat_hbm.at[sidx_v.at[pl.ds((g + 1) * _GB,
                                                            _GB)]],
                                ring_v.at[(g + 1) % 2], sem))
                        cps[g].wait()
                        pltpu.sync_copy(ring_v.at[g % 2],
                                        acc_sh.at[lidx_v.at[g]], add=True)
                    return c

                lax.fori_loop(0, nb, batch, 0)
            plsc.subcore_barrier()
            for wo in range(sub_rows // _GB):
                r0 = sid * sub_rows + wo * _GB
                pltpu.sync_copy(acc_sh.at[pl.ds(r0, _GB)], ring_v.at[0])
                pltpu.sync_copy(ring_v.at[0],
                                out_hbm.at[pl.ds(base + r0, _GB)])
            if ci + 1 < ch_per_core:
                plsc.subcore_barrier()

    return spmm


def _chunk_cfg(nd):
    max_chunk = 26624
    nch = _NC * (-(-nd // (_NC * max_chunk)))
    chunk = -(-(-(-nd // nch)) // (_NS * _GB)) * (_NS * _GB)
    return chunk, nch


def _spmm(part_t, feat, nd, epad):
    chunk, nch = _chunk_cfg(nd)
    psrc, pdst, cnt = part_t
    zeros = jnp.zeros((_GB, _D), jnp.float32)
    out = _spmm_fn(epad, chunk, nch)(psrc, pdst, cnt, feat, zeros)
    return out[:nd]


def _scale_body(x_ref, d_ref, o_ref, *, mode):
    d = d_ref[0, 0, 0, :] + d_ref[1, 0, 0, :]
    if mode == "rsqrt":
        s = 1.0 / (jnp.sqrt(d) + 1e-8)
    else:
        s = 1.0 / (d + 1e-8)
    o_ref[...] = x_ref[...] * s[:, None]


def _scale(x, deg2, mode):
    n = x.shape[0]
    return pl.pallas_call(
        functools.partial(_scale_body, mode=mode),
        grid=(n // _R,),
        in_specs=[
            pl.BlockSpec((_R, _D), lambda i: (i, 0)),
            pl.BlockSpec((_NC, 1, 1, _R), lambda i: (0, i, 0, 0)),
        ],
        out_specs=pl.BlockSpec((_R, _D), lambda i: (i, 0)),
        out_shape=jax.ShapeDtypeStruct((n, _D), jnp.float32),
    )(x, deg2.reshape(_NC, n // _R, 1, _R))


def _post_body(raw_ref, d_ref, acc_ref, scaled_ref, accout_ref, *, div):
    d = d_ref[0, 0, 0, :] + d_ref[1, 0, 0, :]
    inv = 1.0 / (jnp.sqrt(d) + 1e-8)
    cur = raw_ref[...] * inv[:, None] * (1.0 / div)
    scaled_ref[...] = cur * inv[:, None]
    nrm = jnp.maximum(jnp.sqrt(jnp.sum(cur * cur, axis=1, keepdims=True)),
                      1e-12)
    accout_ref[...] = acc_ref[...] + cur / nrm


def _post(raw, deg2, acc, div):
    n = raw.shape[0]
    return pl.pallas_call(
        functools.partial(_post_body, div=float(div)),
        grid=(n // _R,),
        in_specs=[
            pl.BlockSpec((_R, _D), lambda i: (i, 0)),
            pl.BlockSpec((_NC, 1, 1, _R), lambda i: (0, i, 0, 0)),
            pl.BlockSpec((_R, _D), lambda i: (i, 0)),
        ],
        out_specs=[pl.BlockSpec((_R, _D), lambda i: (i, 0))] * 2,
        out_shape=[jax.ShapeDtypeStruct((n, _D), jnp.float32)] * 2,
    )(raw, deg2.reshape(_NC, n // _R, 1, _R), acc)


def _combine_body(a_ref, b_ref, c_ref, o_ref):
    o_ref[...] = 0.5 * a_ref[...] + 0.2 * b_ref[...] + 0.3 * c_ref[...]


def _combine(a, b, c):
    n = a.shape[0]
    return pl.pallas_call(
        _combine_body,
        grid=(n // _R,),
        in_specs=[pl.BlockSpec((_R, _D), lambda i: (i, 0))] * 3,
        out_specs=pl.BlockSpec((_R, _D), lambda i: (i, 0)),
        out_shape=jax.ShapeDtypeStruct((n, _D), jnp.float32),
    )(a, b, c)


def _pad_edges(src, dst, nsrc, epad):
    e = src.shape[0]
    pad = epad - e
    psrc = jnp.concatenate(
        [src, jnp.arange(pad, dtype=jnp.int32) % nsrc])
    pdst = jnp.concatenate(
        [dst, jnp.full((pad,), _DUMMY, jnp.int32)])
    return psrc, pdst


def _propagate(a, b, na, nb, af, bf):
    epad = -(-a.shape[0] // (_NW * _BB)) * (_NW * _BB)
    sfa, dfa = _pad_edges(b, a, nb, epad)
    sfb, dfb = _pad_edges(a, b, na, epad)
    parta = _part(sfa, dfa, na, epad)
    partb = _part(sfb, dfb, nb, epad)
    deg2a = _hist(dfa, na, epad)
    deg2b = _hist(dfb, nb, epad)
    sa = _scale(af, deg2a, "rsqrt")
    sb = _scale(bf, deg2b, "rsqrt")
    acca, accb = af, bf
    for i in range(_L):
        rawa = _spmm(parta, sb, na, epad)
        rawb = _spmm(partb, sa, nb, epad)
        sa, acca = _post(rawa, deg2a, acca, i + 2)
        sb, accb = _post(rawb, deg2b, accb, i + 2)
    return acca, accb, (parta, deg2a, epad)


def kernel(users_feat, bundles_feat, items_feat,
           ub_u, ub_b, ui_u, ui_i, bi_b, bi_i):
    ub_users, ub_bundles, _ = _propagate(
        ub_u, ub_b, _NU, _NB, users_feat, bundles_feat)
    ui_users, ui_items, (ui_pu, ui_degu, ui_epad) = _propagate(
        ui_u, ui_i, _NU, _NI, users_feat, items_feat)
    bi_bundles, bi_items, (bi_pb, bi_degb, bi_epad) = _propagate(
        bi_b, bi_i, _NB, _NI, bundles_feat, items_feat)

    ui_b_raw = _spmm(bi_pb, items_feat, _NB, bi_epad)
    ui_bundles = _scale(ui_b_raw, bi_degb, "recip")
    bi_u_raw = _spmm(ui_pu, bi_items, _NU, ui_epad)
    bi_users = _scale(bi_u_raw, ui_degu, "recip")

    users_rep = _combine(ub_users, ui_users, bi_users)
    bundles_rep = _combine(ub_bundles, ui_bundles, bi_bundles)
    return users_rep, bundles_rep


# speedup vs baseline: 1.0399x; 1.0399x over previous
import functools

import jax
import jax.numpy as jnp
from jax import lax
from jax.experimental import pallas as pl
from jax.experimental.pallas import tpu as pltpu
from jax.experimental.pallas import tpu_sc as plsc

_NU, _NB, _NI, _D = 50000, 20000, 100000, 64
_L = 2

_NC, _NS, _VL = 2, 16, 16
_NW = _NC * _NS
_GB = 128
_BB = 512
_GPB = _BB // _GB
_HC = 8
_R = 400


def _div_le(n, cap):
    for k in range(1, n + 1):
        if n % k == 0 and n // k <= cap:
            return n // k
    return 1


def _mesh():
    return plsc.VectorSubcoreMesh(core_axis_name="c", subcore_axis_name="s")


@functools.lru_cache(maxsize=None)
def _hist_fn(epad, nreal, npad):
    nbatch = epad // (_NW * _BB)
    sub = npad // _NS
    zb = _div_le(sub, 512)

    @functools.partial(
        pl.kernel,
        mesh=_mesh(),
        out_type=jax.ShapeDtypeStruct((_NC, npad, _HC), jnp.float32),
        scratch_types=[
            pltpu.VMEM((_BB,), jnp.int32),
            pltpu.VMEM((_GPB, _GB), jnp.int32),
            pltpu.VMEM((_GB, _HC), jnp.float32),
            pltpu.VMEM((zb, _HC), jnp.float32),
            pltpu.VMEM_SHARED((npad, _HC), jnp.float32),
            pltpu.SemaphoreType.DMA,
        ],
        compiler_params=pltpu.CompilerParams(use_tc_tiling_on_sc=False),
    )
    def hist(idx_hbm, ones_hbm, zeros_hbm, out_hbm,
             idx_v, lidx_v, ones_v, wb_v, cnt_sh, sem):
        del sem
        cid = lax.axis_index("c")
        sid = lax.axis_index("s")
        wid = sid * _NC + cid
        pltpu.sync_copy(ones_hbm, ones_v)
        pltpu.sync_copy(zeros_hbm, wb_v)
        for z in range(sub // zb):
            pltpu.sync_copy(wb_v, cnt_sh.at[pl.ds(sid * sub + z * zb, zb)])
        plsc.subcore_barrier()

        def batch(bi, c):
            off = (wid * nbatch + bi) * _BB
            pltpu.sync_copy(idx_hbm.at[pl.ds(off, _BB)], idx_v)
            for j in range(_BB // _VL):
                v = idx_v[pl.ds(j * _VL, _VL)]
                lv = jnp.where(v < nreal, v, nreal + (v & (_GB - 1)))
                lidx_v[j // (_GB // _VL), pl.ds((j % (_GB // _VL)) * _VL, _VL)] = lv
            for g in range(_GPB):
                pltpu.sync_copy(ones_v, cnt_sh.at[lidx_v.at[g]], add=True)
            return c

        lax.fori_loop(0, nbatch, batch, 0)
        plsc.subcore_barrier()
        for z in range(sub // zb):
            r0 = sid * sub + z * zb
            pltpu.sync_copy(cnt_sh.at[pl.ds(r0, zb)], wb_v)
            pltpu.sync_copy(wb_v, out_hbm.at[cid, pl.ds(r0, zb)])

    return hist


def _hist(dst_flat, nreal, epad):
    npad = -(-(nreal + _GB) // (_NS * _GB)) * (_NS * _GB)
    sub = npad // _NS
    zb = _div_le(sub, 512)
    ones = jnp.zeros((_GB, _HC), jnp.float32).at[:, 0].set(1.0)
    zeros = jnp.zeros((zb, _HC), jnp.float32)
    out = _hist_fn(epad, nreal, npad)(dst_flat, ones, zeros)
    return out[:, :nreal, 0]


_IC = 1024
_ICG = _IC // _GB
_PB = 512
_STG = _IC + _PB + _VL
_DUMMY = 1 << 29


def _cap(epad):
    return -(-(epad // _NW) // _IC) * _IC


@functools.lru_cache(maxsize=None)
def _part_fn(epad, split):
    epw = epad // _NW
    cap = _cap(epad)
    nchk = epw // _PB
    ntot = _NC * _NW * cap

    @functools.partial(
        pl.kernel,
        mesh=_mesh(),
        out_type=[
            jax.ShapeDtypeStruct((ntot,), jnp.int32),
            jax.ShapeDtypeStruct((ntot,), jnp.int32),
            jax.ShapeDtypeStruct((_NW, _VL), jnp.int32),
        ],
        scratch_types=[
            pltpu.VMEM((_PB,), jnp.int32),
            pltpu.VMEM((_PB,), jnp.int32),
            pltpu.VMEM((_STG,), jnp.int32),
            pltpu.VMEM((_STG,), jnp.int32),
            pltpu.VMEM((_STG,), jnp.int32),
            pltpu.VMEM((_STG,), jnp.int32),
            pltpu.VMEM((_VL,), jnp.int32),
            pltpu.SemaphoreType.DMA,
        ],
        compiler_params=pltpu.CompilerParams(use_tc_tiling_on_sc=False,
                                             needs_layout_passes=False),
    )
    def part(src_hbm, dst_hbm, psrc_hbm, pdst_hbm, cnt_hbm,
             src_v, dst_v, ss0, sd0, ss1, sd1, cnt_v, sem):
        del sem
        cid = lax.axis_index("c")
        sid = lax.axis_index("s")
        wid = sid * _NC + cid
        e0 = wid * epw
        lanes = lax.broadcasted_iota(jnp.int32, (_VL,), 0)

        def flush(bkt, ss, sd, cur, nf):
            regbase = (bkt * _NW + wid) * cap
            pltpu.sync_copy(ss.at[pl.ds(0, _IC)],
                            psrc_hbm.at[pl.ds(regbase + nf * _IC, _IC)])
            pltpu.sync_copy(sd.at[pl.ds(0, _IC)],
                            pdst_hbm.at[pl.ds(regbase + nf * _IC, _IC)])
            r = cur - _IC
            for g in range((_STG - _IC) // _VL):
                k = r - g * _VL
                vs = jnp.where(lanes < k, ss[pl.ds(_IC + g * _VL, _VL)], 0)
                vd = jnp.where(lanes < k, sd[pl.ds(_IC + g * _VL, _VL)],
                               _DUMMY)
                ss[pl.ds(g * _VL, _VL)] = vs
                sd[pl.ds(g * _VL, _VL)] = vd
            return r, nf + 1

        def maybe_flush(bkt, ss, sd, cur, nf):
            return lax.cond(
                cur >= _IC,
                lambda c, n: flush(bkt, ss, sd, c, n),
                lambda c, n: (c, n),
                cur, nf)

        def chunkbody(ki, carry):
            cur0, cur1, nf0, nf1 = carry
            off = e0 + ki * _PB
            pltpu.sync_copy(src_hbm.at[pl.ds(off, _PB)], src_v)
            pltpu.sync_copy(dst_hbm.at[pl.ds(off, _PB)], dst_v)
            for j in range(_PB // _VL):
                vs = src_v[pl.ds(j * _VL, _VL)]
                vd = dst_v[pl.ds(j * _VL, _VL)]
                m0 = vd < split
                m1 = (vd >= split) & (vd < _DUMMY)
                key0 = jnp.where(m0, 0, _VL) + lanes
                key1 = jnp.where(m1, 0, _VL) + lanes
                _, s0v = plsc.sort_key_val(key0, vs)
                _, d0v = plsc.sort_key_val(key0, vd)
                ss0[pl.ds(cur0, _VL)] = s0v
                sd0[pl.ds(cur0, _VL)] = d0v
                cur0 = cur0 + plsc.all_reduce_population_count(m0)[0]
                _, s1v = plsc.sort_key_val(key1, vs)
                _, d1v = plsc.sort_key_val(key1, vd)
                ss1[pl.ds(cur1, _VL)] = s1v
                sd1[pl.ds(cur1, _VL)] = d1v
                cur1 = cur1 + plsc.all_reduce_population_count(m1)[0]
            cur0, nf0 = maybe_flush(0, ss0, sd0, cur0, nf0)
            cur1, nf1 = maybe_flush(1, ss1, sd1, cur1, nf1)
            return cur0, cur1, nf0, nf1

        cur0, cur1, nf0, nf1 = lax.fori_loop(
            0, nchk, chunkbody, (jnp.int32(0), jnp.int32(0),
                                 jnp.int32(0), jnp.int32(0)))

        for g in range(_STG // _VL):
            k0 = cur0 - g * _VL
            ss0[pl.ds(g * _VL, _VL)] = jnp.where(
                lanes < k0, ss0[pl.ds(g * _VL, _VL)], 0)
            sd0[pl.ds(g * _VL, _VL)] = jnp.where(
                lanes < k0, sd0[pl.ds(g * _VL, _VL)], _DUMMY)
            k1 = cur1 - g * _VL
            ss1[pl.ds(g * _VL, _VL)] = jnp.where(
                lanes < k1, ss1[pl.ds(g * _VL, _VL)], 0)
            sd1[pl.ds(g * _VL, _VL)] = jnp.where(
                lanes < k1, sd1[pl.ds(g * _VL, _VL)], _DUMMY)
        _, nf0 = lax.cond(
            cur0 > 0,
            lambda c, n: flush(0, ss0, sd0, c, n),
            lambda c, n: (c, n), cur0 + _IC, nf0)
        _, nf1 = lax.cond(
            cur1 > 0,
            lambda c, n: flush(1, ss1, sd1, c, n),
            lambda c, n: (c, n), cur1 + _IC, nf1)

        cnt_v[pl.ds(0, _VL)] = (jnp.where(lanes == 0, nf0, 0)
                                + jnp.where(lanes == 1, nf1, 0))
        pltpu.sync_copy(cnt_v, cnt_hbm.at[wid])

    return part


def _part(src_flat, dst_flat, nd, epad):
    chunk, nch = _chunk_cfg(nd)
    split = (nch // _NC) * chunk
    psrc, pdst, cnt = _part_fn(epad, split)(src_flat, dst_flat)
    return psrc, pdst, cnt.reshape(-1)


@functools.lru_cache(maxsize=None)
def _spmm_fn(epad, chunk, nch):
    cap = _cap(epad)
    ch_per_core = nch // _NC
    sub_rows = chunk // _NS

    @functools.partial(
        pl.kernel,
        mesh=_mesh(),
        out_type=jax.ShapeDtypeStruct((nch * chunk, _D), jnp.float32),
        scratch_types=[
            pltpu.VMEM((_IC,), jnp.int32),
            pltpu.VMEM((_IC,), jnp.int32),
            pltpu.VMEM((_ICG, _GB), jnp.int32),
            pltpu.VMEM((2, _GB, _D), jnp.float32),
            pltpu.VMEM((_NW * _VL + _VL,), jnp.int32),
            pltpu.VMEM_SHARED((chunk + _GB, _D), jnp.float32),
            pltpu.SemaphoreType.DMA,
        ],
        compiler_params=pltpu.CompilerParams(use_tc_tiling_on_sc=False),
    )
    def spmm(psrc_hbm, pdst_hbm, cnt_hbm, feat_hbm, zeros_hbm, out_hbm,
             sidx_v, didx_v, lidx_v, ring_v, cnt_v, acc_sh, sem):
        cid = lax.axis_index("c")
        sid = lax.axis_index("s")
        pltpu.sync_copy(cnt_hbm, cnt_v.at[pl.ds(0, _NW * _VL)])
        for ci in range(ch_per_core):
            chunk_id = cid * ch_per_core + ci
            base = chunk_id * chunk
            pltpu.sync_copy(zeros_hbm, ring_v.at[0])
            for z in range(sub_rows // _GB):
                pltpu.sync_copy(
                    ring_v.at[0],
                    acc_sh.at[pl.ds(sid * sub_rows + z * _GB, _GB)])
            plsc.subcore_barrier()

            for reg in range(_NW // _NS):
                w = sid * (_NW // _NS) + reg
                nb = cnt_v[pl.ds(w * _VL + cid, _VL)][0]
                regbase = (cid * _NW + w) * cap

                def batch(bi, c):
                    off = regbase + bi * _IC
                    pltpu.sync_copy(psrc_hbm.at[pl.ds(off, _IC)], sidx_v)
                    pltpu.sync_copy(pdst_hbm.at[pl.ds(off, _IC)], didx_v)
                    cps = [pltpu.async_copy(
                        feat_hbm.at[sidx_v.at[pl.ds(0, _GB)]],
                        ring_v.at[0], sem)]
                    for j in range(_IC // _VL):
                        v = didx_v[pl.ds(j * _VL, _VL)]
                        loc = v - base
                        ok = (loc >= 0) & (loc < chunk)
                        lv = jnp.where(ok, loc, chunk + (v & (_GB - 1)))
                        lidx_v[j // (_GB // _VL),
                               pl.ds((j % (_GB // _VL)) * _VL, _VL)] = lv
                    for g in range(_ICG):
                        if g + 1 < _ICG:
                            cps.append(pltpu.async_copy(
                                feat_hbm.at[sidx_v.at[pl.ds((g + 1) * _GB,
                                                            _GB)]],
                                ring_v.at[(g + 1) % 2], sem))
                        cps[g].wait()
                        pltpu.sync_copy(ring_v.at[g % 2],
                                        acc_sh.at[lidx_v.at[g]], add=True)
                    return c

                lax.fori_loop(0, nb, batch, 0)
            plsc.subcore_barrier()
            for wo in range(sub_rows // _GB):
                r0 = sid * sub_rows + wo * _GB
                pltpu.sync_copy(acc_sh.at[pl.ds(r0, _GB)], ring_v.at[0])
                pltpu.sync_copy(ring_v.at[0],
                                out_hbm.at[pl.ds(base + r0, _GB)])
            if ci + 1 < ch_per_core:
                plsc.subcore_barrier()

    return spmm


def _chunk_cfg(nd):
    max_chunk = 26624
    nch = _NC * (-(-nd // (_NC * max_chunk)))
    chunk = -(-(-(-nd // nch)) // (_NS * _GB)) * (_NS * _GB)
    return chunk, nch


def _spmm(part_t, feat, nd, epad):
    chunk, nch = _chunk_cfg(nd)
    psrc, pdst, cnt = part_t
    zeros = jnp.zeros((_GB, _D), jnp.float32)
    out = _spmm_fn(epad, chunk, nch)(psrc, pdst, cnt, feat, zeros)
    return out[:nd]


def _scale_body(x_ref, d_ref, o_ref, *, mode):
    d = d_ref[0, 0, 0, :] + d_ref[1, 0, 0, :]
    if mode == "rsqrt":
        s = 1.0 / (jnp.sqrt(d) + 1e-8)
    else:
        s = 1.0 / (d + 1e-8)
    o_ref[...] = x_ref[...] * s[:, None]


def _scale(x, deg2, mode):
    n = x.shape[0]
    return pl.pallas_call(
        functools.partial(_scale_body, mode=mode),
        grid=(n // _R,),
        in_specs=[
            pl.BlockSpec((_R, _D), lambda i: (i, 0)),
            pl.BlockSpec((_NC, 1, 1, _R), lambda i: (0, i, 0, 0)),
        ],
        out_specs=pl.BlockSpec((_R, _D), lambda i: (i, 0)),
        out_shape=jax.ShapeDtypeStruct((n, _D), jnp.float32),
    )(x, deg2.reshape(_NC, n // _R, 1, _R))


def _post_body(raw_ref, d_ref, acc_ref, scaled_ref, accout_ref, *, div):
    d = d_ref[0, 0, 0, :] + d_ref[1, 0, 0, :]
    inv = 1.0 / (jnp.sqrt(d) + 1e-8)
    cur = raw_ref[...] * inv[:, None] * (1.0 / div)
    scaled_ref[...] = cur * inv[:, None]
    nrm = jnp.maximum(jnp.sqrt(jnp.sum(cur * cur, axis=1, keepdims=True)),
                      1e-12)
    accout_ref[...] = acc_ref[...] + cur / nrm


def _post(raw, deg2, acc, div):
    n = raw.shape[0]
    return pl.pallas_call(
        functools.partial(_post_body, div=float(div)),
        grid=(n // _R,),
        in_specs=[
            pl.BlockSpec((_R, _D), lambda i: (i, 0)),
            pl.BlockSpec((_NC, 1, 1, _R), lambda i: (0, i, 0, 0)),
            pl.BlockSpec((_R, _D), lambda i: (i, 0)),
        ],
        out_specs=[pl.BlockSpec((_R, _D), lambda i: (i, 0))] * 2,
        out_shape=[jax.ShapeDtypeStruct((n, _D), jnp.float32)] * 2,
    )(raw, deg2.reshape(_NC, n // _R, 1, _R), acc)


def _combine_body(a_ref, b_ref, c_ref, o_ref):
    o_ref[...] = 0.5 * a_ref[...] + 0.2 * b_ref[...] + 0.3 * c_ref[...]


def _combine(a, b, c):
    n = a.shape[0]
    return pl.pallas_call(
        _combine_body,
        grid=(n // _R,),
        in_specs=[pl.BlockSpec((_R, _D), lambda i: (i, 0))] * 3,
        out_specs=pl.BlockSpec((_R, _D), lambda i: (i, 0)),
        out_shape=jax.ShapeDtypeStruct((n, _D), jnp.float32),
    )(a, b, c)


def _pad_edges(src, dst, nsrc, epad):
    e = src.shape[0]
    pad = epad - e
    psrc = jnp.concatenate(
        [src, jnp.arange(pad, dtype=jnp.int32) % nsrc])
    pdst = jnp.concatenate(
        [dst, jnp.full((pad,), _DUMMY, jnp.int32)])
    return psrc, pdst


def _propagate(a, b, na, nb, af, bf):
    epad = -(-a.shape[0] // (_NW * _BB)) * (_NW * _BB)
    sfa, dfa = _pad_edges(b, a, nb, epad)
    sfb, dfb = _pad_edges(a, b, na, epad)
    parta = _part(sfa, dfa, na, epad)
    partb = _part(sfb, dfb, nb, epad)
    deg2a = _hist(dfa, na, epad)
    deg2b = _hist(dfb, nb, epad)
    sa = _scale(af, deg2a, "rsqrt")
    sb = _scale(bf, deg2b, "rsqrt")
    acca, accb = af, bf
    for i in range(_L):
        rawa = _spmm(parta, sb, na, epad)
        rawb = _spmm(partb, sa, nb, epad)
        sa, acca = _post(rawa, deg2a, acca, i + 2)
        sb, accb = _post(rawb, deg2b, accb, i + 2)
    return acca, accb, (parta, deg2a, epad)


def kernel(users_feat, bundles_feat, items_feat,
           ub_u, ub_b, ui_u, ui_i, bi_b, bi_i):
    ub_users, ub_bundles, _ = _propagate(
        ub_u, ub_b, _NU, _NB, users_feat, bundles_feat)
    ui_users, ui_items, (ui_pu, ui_degu, ui_epad) = _propagate(
        ui_u, ui_i, _NU, _NI, users_feat, items_feat)
    bi_bundles, bi_items, (bi_pb, bi_degb, bi_epad) = _propagate(
        bi_b, bi_i, _NB, _NI, bundles_feat, items_feat)

    ui_b_raw = _spmm(bi_pb, items_feat, _NB, bi_epad)
    ui_bundles = _scale(ui_b_raw, bi_degb, "recip")
    bi_u_raw = _spmm(ui_pu, bi_items, _NU, ui_epad)
    bi_users = _scale(bi_u_raw, ui_degu, "recip")

    users_rep = _combine(ub_users, ui_users, bi_users)
    bundles_rep = _combine(ub_bundles, ui_bundles, bi_bundles)
    return users_rep, bundles_rep


# speedup vs baseline: 2.7125x; 2.6084x over previous
import functools

import jax
import jax.numpy as jnp
from jax import lax
from jax.experimental import pallas as pl
from jax.experimental.pallas import tpu as pltpu
from jax.experimental.pallas import tpu_sc as plsc

_NU, _NB, _NI, _D = 50000, 20000, 100000, 64
_L = 2

_NC, _NS, _VL = 2, 16, 16
_NW = _NC * _NS
_GB = 128
_BB = 512
_GPB = _BB // _GB
_HC = 8
_R = 400


def _div_le(n, cap):
    for k in range(1, n + 1):
        if n % k == 0 and n // k <= cap:
            return n // k
    return 1


def _mesh():
    return plsc.VectorSubcoreMesh(core_axis_name="c", subcore_axis_name="s")


@functools.lru_cache(maxsize=None)
def _hist_fn(epad, nreal, npad):
    nbatch = epad // (_NW * _BB)
    sub = npad // _NS
    zb = _div_le(sub, 512)

    @functools.partial(
        pl.kernel,
        mesh=_mesh(),
        out_type=jax.ShapeDtypeStruct((_NC, npad, _HC), jnp.float32),
        scratch_types=[
            pltpu.VMEM((_BB,), jnp.int32),
            pltpu.VMEM((_GPB, _GB), jnp.int32),
            pltpu.VMEM((_GB, _HC), jnp.float32),
            pltpu.VMEM((zb, _HC), jnp.float32),
            pltpu.VMEM_SHARED((npad, _HC), jnp.float32),
            pltpu.SemaphoreType.DMA,
        ],
        compiler_params=pltpu.CompilerParams(use_tc_tiling_on_sc=False),
    )
    def hist(idx_hbm, ones_hbm, zeros_hbm, out_hbm,
             idx_v, lidx_v, ones_v, wb_v, cnt_sh, sem):
        del sem
        cid = lax.axis_index("c")
        sid = lax.axis_index("s")
        wid = sid * _NC + cid
        pltpu.sync_copy(ones_hbm, ones_v)
        pltpu.sync_copy(zeros_hbm, wb_v)
        for z in range(sub // zb):
            pltpu.sync_copy(wb_v, cnt_sh.at[pl.ds(sid * sub + z * zb, zb)])
        plsc.subcore_barrier()

        def batch(bi, c):
            off = (wid * nbatch + bi) * _BB
            pltpu.sync_copy(idx_hbm.at[pl.ds(off, _BB)], idx_v)
            for j in range(_BB // _VL):
                v = idx_v[pl.ds(j * _VL, _VL)]
                lv = jnp.where(v < nreal, v, nreal + (v & (_GB - 1)))
                lidx_v[j // (_GB // _VL), pl.ds((j % (_GB // _VL)) * _VL, _VL)] = lv
            for g in range(_GPB):
                pltpu.sync_copy(ones_v, cnt_sh.at[lidx_v.at[g]], add=True)
            return c

        lax.fori_loop(0, nbatch, batch, 0)
        plsc.subcore_barrier()
        for z in range(sub // zb):
            r0 = sid * sub + z * zb
            pltpu.sync_copy(cnt_sh.at[pl.ds(r0, zb)], wb_v)
            pltpu.sync_copy(wb_v, out_hbm.at[cid, pl.ds(r0, zb)])

    return hist


def _hist(dst_flat, nreal, epad):
    npad = -(-(nreal + _GB) // (_NS * _GB)) * (_NS * _GB)
    sub = npad // _NS
    zb = _div_le(sub, 512)
    ones = jnp.zeros((_GB, _HC), jnp.float32).at[:, 0].set(1.0)
    zeros = jnp.zeros((zb, _HC), jnp.float32)
    out = _hist_fn(epad, nreal, npad)(dst_flat, ones, zeros)
    return out[:, :nreal, 0]


_IC = 1024
_ICG = _IC // _GB


@functools.lru_cache(maxsize=None)
def _spmm_fn(epad, chunk, nch):
    nbatch = epad // (_NS * _IC)
    ch_per_core = nch // _NC
    sub_rows = chunk // _NS

    @functools.partial(
        pl.kernel,
        mesh=_mesh(),
        out_type=jax.ShapeDtypeStruct((nch * chunk, _D), jnp.float32),
        scratch_types=[
            pltpu.VMEM((2, _ICG, _GB), jnp.int32),
            pltpu.VMEM((2, _IC), jnp.int32),
            pltpu.VMEM((_ICG, _GB), jnp.int32),
            pltpu.VMEM((2, _GB, _D), jnp.float32),
            pltpu.VMEM_SHARED((chunk + _GB, _D), jnp.float32),
            pltpu.SemaphoreType.DMA,
            pltpu.SemaphoreType.DMA,
        ],
        compiler_params=pltpu.CompilerParams(use_tc_tiling_on_sc=False),
    )
    def spmm(sidx_hbm, didx_hbm, feat_hbm, zeros_hbm, out_hbm,
             sidx_v, didx_v, lidx_v, ring_v, acc_sh, sem, isem):
        cid = lax.axis_index("c")
        sid = lax.axis_index("s")

        def fire_idx(bi, buf):
            row0 = (sid * nbatch + bi) * _ICG
            pltpu.async_copy(sidx_hbm.at[pl.ds(row0, _ICG)],
                             sidx_v.at[buf], isem)
            pltpu.async_copy(didx_hbm.at[pl.ds(row0 * _GB, _IC)],
                             didx_v.at[buf], isem)

        def drain_idx(buf):
            pltpu.make_async_copy(sidx_hbm.at[pl.ds(0, _ICG)],
                                  sidx_v.at[buf], isem).wait()
            pltpu.make_async_copy(didx_hbm.at[pl.ds(0, _IC)],
                                  didx_v.at[buf], isem).wait()

        for ci in range(ch_per_core):
            chunk_id = cid * ch_per_core + ci
            base = chunk_id * chunk
            fire_idx(0, 0)
            pltpu.sync_copy(zeros_hbm, ring_v.at[0])
            for z in range(sub_rows // _GB):
                pltpu.sync_copy(
                    ring_v.at[0],
                    acc_sh.at[pl.ds(sid * sub_rows + z * _GB, _GB)])
            plsc.subcore_barrier()

            def batch(bi, c):
                b = lax.rem(bi, 2)
                drain_idx(b)
                cps = [pltpu.async_copy(feat_hbm.at[sidx_v.at[b, 0]],
                                        ring_v.at[0], sem)]
                fire_idx(jnp.minimum(bi + 1, nbatch - 1), 1 - b)
                for j in range(_IC // _VL):
                    v = didx_v[b, pl.ds(j * _VL, _VL)]
                    loc = v - base
                    ok = (loc >= 0) & (loc < chunk)
                    lv = jnp.where(ok, loc, chunk + (v & (_GB - 1)))
                    lidx_v[j // (_GB // _VL),
                           pl.ds((j % (_GB // _VL)) * _VL, _VL)] = lv
                for g in range(_ICG):
                    if g + 1 < _ICG:
                        cps.append(
                            pltpu.async_copy(feat_hbm.at[sidx_v.at[b, g + 1]],
                                             ring_v.at[(g + 1) % 2], sem))
                    cps[g].wait()
                    pltpu.sync_copy(ring_v.at[g % 2],
                                    acc_sh.at[lidx_v.at[g]], add=True)
                return c

            lax.fori_loop(0, nbatch, batch, 0)
            drain_idx(0)
            plsc.subcore_barrier()
            for w in range(sub_rows // _GB):
                r0 = sid * sub_rows + w * _GB
                pltpu.sync_copy(acc_sh.at[pl.ds(r0, _GB)], ring_v.at[0])
                pltpu.sync_copy(ring_v.at[0],
                                out_hbm.at[pl.ds(base + r0, _GB)])
            if ci + 1 < ch_per_core:
                plsc.subcore_barrier()

    return spmm


def _chunk_cfg(nd):
    max_chunk = 26624
    nch = _NC * (-(-nd // (_NC * max_chunk)))
    chunk = -(-(-(-nd // nch)) // (_NS * _GB)) * (_NS * _GB)
    return chunk, nch


def _spmm(src2d, dst_flat, feat, nd, epad):
    chunk, nch = _chunk_cfg(nd)
    zeros = jnp.zeros((_GB, _D), jnp.float32)
    out = _spmm_fn(epad, chunk, nch)(src2d, dst_flat, feat, zeros)
    return out[:nd]


def _scale_body(x_ref, d_ref, o_ref, *, mode):
    d = d_ref[0, 0, 0, :] + d_ref[1, 0, 0, :]
    if mode == "rsqrt":
        s = 1.0 / (jnp.sqrt(d) + 1e-8)
    else:
        s = 1.0 / (d + 1e-8)
    o_ref[...] = x_ref[...] * s[:, None]


def _scale(x, deg2, mode):
    n = x.shape[0]
    return pl.pallas_call(
        functools.partial(_scale_body, mode=mode),
        grid=(n // _R,),
        in_specs=[
            pl.BlockSpec((_R, _D), lambda i: (i, 0)),
            pl.BlockSpec((_NC, 1, 1, _R), lambda i: (0, i, 0, 0)),
        ],
        out_specs=pl.BlockSpec((_R, _D), lambda i: (i, 0)),
        out_shape=jax.ShapeDtypeStruct((n, _D), jnp.float32),
    )(x, deg2.reshape(_NC, n // _R, 1, _R))


def _post_body(raw_ref, d_ref, acc_ref, scaled_ref, accout_ref, *, div):
    d = d_ref[0, 0, 0, :] + d_ref[1, 0, 0, :]
    inv = 1.0 / (jnp.sqrt(d) + 1e-8)
    cur = raw_ref[...] * inv[:, None] * (1.0 / div)
    scaled_ref[...] = cur * inv[:, None]
    nrm = jnp.maximum(jnp.sqrt(jnp.sum(cur * cur, axis=1, keepdims=True)),
                      1e-12)
    accout_ref[...] = acc_ref[...] + cur / nrm


def _post(raw, deg2, acc, div):
    n = raw.shape[0]
    return pl.pallas_call(
        functools.partial(_post_body, div=float(div)),
        grid=(n // _R,),
        in_specs=[
            pl.BlockSpec((_R, _D), lambda i: (i, 0)),
            pl.BlockSpec((_NC, 1, 1, _R), lambda i: (0, i, 0, 0)),
            pl.BlockSpec((_R, _D), lambda i: (i, 0)),
        ],
        out_specs=[pl.BlockSpec((_R, _D), lambda i: (i, 0))] * 2,
        out_shape=[jax.ShapeDtypeStruct((n, _D), jnp.float32)] * 2,
    )(raw, deg2.reshape(_NC, n // _R, 1, _R), acc)


def _combine_body(a_ref, b_ref, c_ref, o_ref):
    o_ref[...] = 0.5 * a_ref[...] + 0.2 * b_ref[...] + 0.3 * c_ref[...]


def _combine(a, b, c):
    n = a.shape[0]
    return pl.pallas_call(
        _combine_body,
        grid=(n // _R,),
        in_specs=[pl.BlockSpec((_R, _D), lambda i: (i, 0))] * 3,
        out_specs=pl.BlockSpec((_R, _D), lambda i: (i, 0)),
        out_shape=jax.ShapeDtypeStruct((n, _D), jnp.float32),
    )(a, b, c)


def _pad_edges(src, dst, nsrc, epad):
    e = src.shape[0]
    pad = epad - e
    psrc = jnp.concatenate(
        [src, jnp.arange(pad, dtype=jnp.int32) % nsrc])
    pdst = jnp.concatenate(
        [dst, jnp.full((pad,), 1 << 30, jnp.int32)])
    return psrc.reshape(epad // _GB, _GB), pdst


def _propagate(a, b, na, nb, af, bf):
    epad = -(-a.shape[0] // (_NW * _BB)) * (_NW * _BB)
    b_s, a_d = _pad_edges(b, a, nb, epad)
    a_s, b_d = _pad_edges(a, b, na, epad)
    deg2a = _hist(a_d, na, epad)
    deg2b = _hist(b_d, nb, epad)
    sa = _scale(af, deg2a, "rsqrt")
    sb = _scale(bf, deg2b, "rsqrt")
    acca, accb = af, bf
    for i in range(_L):
        rawa = _spmm(b_s, a_d, sb, na, epad)
        rawb = _spmm(a_s, b_d, sa, nb, epad)
        sa, acca = _post(rawa, deg2a, acca, i + 2)
        sb, accb = _post(rawb, deg2b, accb, i + 2)
    return acca, accb, (b_s, a_d, deg2a, epad)


def kernel(users_feat, bundles_feat, items_feat,
           ub_u, ub_b, ui_u, ui_i, bi_b, bi_i):
    ub_users, ub_bundles, _ = _propagate(
        ub_u, ub_b, _NU, _NB, users_feat, bundles_feat)
    ui_users, ui_items, (ui_is, ui_ud, ui_degu, ui_epad) = _propagate(
        ui_u, ui_i, _NU, _NI, users_feat, items_feat)
    bi_bundles, bi_items, (bi_is, bi_bd, bi_degb, bi_epad) = _propagate(
        bi_b, bi_i, _NB, _NI, bundles_feat, items_feat)

    ui_b_raw = _spmm(bi_is, bi_bd, items_feat, _NB, bi_epad)
    ui_bundles = _scale(ui_b_raw, bi_degb, "recip")
    bi_u_raw = _spmm(ui_is, ui_ud, bi_items, _NU, ui_epad)
    bi_users = _scale(bi_u_raw, ui_degu, "recip")

    users_rep = _combine(ub_users, ui_users, bi_users)
    bundles_rep = _combine(ub_bundles, ui_bundles, bi_bundles)
    return users_rep, bundles_rep
